# Initial kernel scaffold; baseline (speedup 1.0000x reference)
import jax
import jax.numpy as jnp
from jax.experimental import pallas as pl


def kernel(x, incidence, enc_W1, enc_b1, enc_g, enc_be, enc_W2, enc_b2, conv_W, dec_W1, dec_b1, dec_g, dec_be, dec_W2, dec_b2):
    raise NotImplementedError("write your pallas kernel here")


# speedup vs baseline: 1.9183x; 1.9183x over previous
import jax
import jax.numpy as jnp
from jax.experimental import pallas as pl

_N = 10000
_D = 256

_BM_E = 1000
_BM = 400


def _layer_norm(h, g, b, eps=1e-5):
    mu = jnp.mean(h, axis=-1, keepdims=True)
    var = jnp.mean((h - mu) ** 2, axis=-1, keepdims=True)
    return (h - mu) / jnp.sqrt(var + eps) * g + b


def _encode_body(x_ref, w1_ref, b1_ref, g_ref, be_ref, w2_ref, b2_ref,
                 cw_ref, xm_ref):
    h = jnp.dot(x_ref[...], w1_ref[...], preferred_element_type=jnp.float32)
    h = jnp.maximum(h + b1_ref[...], 0.0)
    h = _layer_norm(h, g_ref[...], be_ref[...])
    h = jnp.dot(h, w2_ref[...], preferred_element_type=jnp.float32)
    h = jnp.maximum(h + b2_ref[...], 0.0)
    xm_ref[...] = jnp.dot(h, cw_ref[...], preferred_element_type=jnp.float32)


def _conv_body(inc_ref, xm_ref, w1_ref, b1_ref, g_ref, be_ref, w2_ref, b2_ref,
               out_ref):
    blk = inc_ref[...]
    acc = jnp.dot(blk, xm_ref[...], preferred_element_type=jnp.float32)
    rs = jnp.sum(blk, axis=1, keepdims=True)
    xt = acc / rs
    h = jnp.dot(xt, w1_ref[...], preferred_element_type=jnp.float32)
    h = jnp.maximum(h + b1_ref[...], 0.0)
    h = _layer_norm(h, g_ref[...], be_ref[...])
    h = jnp.dot(h, w2_ref[...], preferred_element_type=jnp.float32)
    out_ref[...] = jnp.maximum(h + b2_ref[...], 0.0)


def kernel(x, incidence, enc_W1, enc_b1, enc_g, enc_be, enc_W2, enc_b2,
           conv_W, dec_W1, dec_b1, dec_g, dec_be, dec_W2, dec_b2):
    f32 = jnp.float32
    ew1, ew2 = enc_W1.T, enc_W2.T
    dw1, dw2 = dec_W1.T, dec_W2.T
    eb1, eb2 = enc_b1.reshape(1, _D), enc_b2.reshape(1, _D)
    db1, db2 = dec_b1.reshape(1, _D), dec_b2.reshape(1, _D)
    eg, ebe = enc_g.reshape(1, _D), enc_be.reshape(1, _D)
    dg, dbe = dec_g.reshape(1, _D), dec_be.reshape(1, _D)

    wspec = pl.BlockSpec((_D, _D), lambda i: (0, 0))
    vspec = pl.BlockSpec((1, _D), lambda i: (0, 0))
    xm = pl.pallas_call(
        _encode_body,
        grid=(_N // _BM_E,),
        in_specs=[pl.BlockSpec((_BM_E, _D), lambda i: (i, 0)),
                  wspec, vspec, vspec, vspec, wspec, vspec, wspec],
        out_specs=pl.BlockSpec((_BM_E, _D), lambda i: (i, 0)),
        out_shape=jax.ShapeDtypeStruct((_N, _D), f32),
    )(x, ew1, eb1, eg, ebe, ew2, eb2, conv_W)

    out = pl.pallas_call(
        _conv_body,
        grid=(_N // _BM,),
        in_specs=[pl.BlockSpec((_BM, _N), lambda i: (i, 0)),
                  pl.BlockSpec((_N, _D), lambda i: (0, 0)),
                  wspec, vspec, vspec, vspec, wspec, vspec],
        out_specs=pl.BlockSpec((_BM, _D), lambda i: (i, 0)),
        out_shape=jax.ShapeDtypeStruct((_N, _D), f32),
    )(incidence, xm, dw1, db1, dg, dbe, dw2, db2)
    return out


# speedup vs baseline: 1.9469x; 1.0149x over previous
import jax
import jax.numpy as jnp
from jax.experimental import pallas as pl

_N = 10000
_D = 256

_BM_E = 1000
_BM = 400


def _layer_norm(h, g, b, eps=1e-5):
    mu = jnp.mean(h, axis=-1, keepdims=True)
    var = jnp.mean((h - mu) ** 2, axis=-1, keepdims=True)
    return (h - mu) / jnp.sqrt(var + eps) * g + b


def _encode_body(x_ref, w1_ref, b1_ref, g_ref, be_ref, w2_ref, b2_ref,
                 cw_ref, xm_ref):
    h = jnp.dot(x_ref[...], w1_ref[...], preferred_element_type=jnp.float32)
    h = jnp.maximum(h + b1_ref[...], 0.0)
    h = _layer_norm(h, g_ref[...], be_ref[...])
    h = jnp.dot(h, w2_ref[...], preferred_element_type=jnp.float32)
    h = jnp.maximum(h + b2_ref[...], 0.0)
    xm_ref[...] = jnp.dot(
        h, cw_ref[...], preferred_element_type=jnp.float32
    ).astype(jnp.bfloat16)


def _conv_body(inc_ref, xm_ref, w1_ref, b1_ref, g_ref, be_ref, w2_ref, b2_ref,
               out_ref):
    blk = inc_ref[...]
    acc = jnp.dot(blk.astype(jnp.bfloat16), xm_ref[...],
                  preferred_element_type=jnp.float32)
    rs = jnp.sum(blk, axis=1, keepdims=True)
    xt = acc / rs
    h = jnp.dot(xt, w1_ref[...], preferred_element_type=jnp.float32)
    h = jnp.maximum(h + b1_ref[...], 0.0)
    h = _layer_norm(h, g_ref[...], be_ref[...])
    h = jnp.dot(h, w2_ref[...], preferred_element_type=jnp.float32)
    out_ref[...] = jnp.maximum(h + b2_ref[...], 0.0)


def kernel(x, incidence, enc_W1, enc_b1, enc_g, enc_be, enc_W2, enc_b2,
           conv_W, dec_W1, dec_b1, dec_g, dec_be, dec_W2, dec_b2):
    f32 = jnp.float32
    ew1, ew2 = enc_W1.T, enc_W2.T
    dw1, dw2 = dec_W1.T, dec_W2.T
    eb1, eb2 = enc_b1.reshape(1, _D), enc_b2.reshape(1, _D)
    db1, db2 = dec_b1.reshape(1, _D), dec_b2.reshape(1, _D)
    eg, ebe = enc_g.reshape(1, _D), enc_be.reshape(1, _D)
    dg, dbe = dec_g.reshape(1, _D), dec_be.reshape(1, _D)

    wspec = pl.BlockSpec((_D, _D), lambda i: (0, 0))
    vspec = pl.BlockSpec((1, _D), lambda i: (0, 0))
    xm = pl.pallas_call(
        _encode_body,
        grid=(_N // _BM_E,),
        in_specs=[pl.BlockSpec((_BM_E, _D), lambda i: (i, 0)),
                  wspec, vspec, vspec, vspec, wspec, vspec, wspec],
        out_specs=pl.BlockSpec((_BM_E, _D), lambda i: (i, 0)),
        out_shape=jax.ShapeDtypeStruct((_N, _D), jnp.bfloat16),
    )(x, ew1, eb1, eg, ebe, ew2, eb2, conv_W)

    out = pl.pallas_call(
        _conv_body,
        grid=(_N // _BM,),
        in_specs=[pl.BlockSpec((_BM, _N), lambda i: (i, 0)),
                  pl.BlockSpec((_N, _D), lambda i: (0, 0)),
                  wspec, vspec, vspec, vspec, wspec, vspec],
        out_specs=pl.BlockSpec((_BM, _D), lambda i: (i, 0)),
        out_shape=jax.ShapeDtypeStruct((_N, _D), f32),
    )(incidence, xm, dw1, db1, dg, dbe, dw2, db2)
    return out
